# baseline (speedup 1.0000x reference)
import functools

import jax
import jax.numpy as jnp
from jax import lax
from jax.experimental import pallas as pl
from jax.experimental.pallas import tpu as pltpu
from jax.experimental.pallas import tpu_sc as plsc

N_CORES = 2
N_SUBCORES = 16
C = 112
S = 10
D = 128
LANES = 16
K0 = 11
K1 = 17
K_TOT = N_SUBCORES * (K0 + K1)


def _worker(feat_hbm, idx3_hbm, out_hbm, idx_v, acc_v, out_v, isem, gsem,
            osem, base, K):
    zeros = jnp.zeros((LANES,), jnp.float32)
    inv = jnp.full((LANES,), 1.0 / S, jnp.float32)

    idx_d = [None] * K
    gat_d = [None] * K
    out_d = [None] * K

    def load_idx(g):
        idx_d[g] = pltpu.async_copy(idx3_hbm.at[base + g], idx_v.at[g % 2],
                                    isem)

    def fire_gathers(g):
        idx_d[g].wait()
        gat_d[g] = [
            pltpu.async_copy(feat_hbm.at[idx_v.at[g % 2].at[si]],
                             acc_v.at[g % 2], gsem, add=True)
            for si in range(S)
        ]

    def compute_store(g):
        p = g % 2
        if g >= 2:
            out_d[g - 2].wait()

        @pl.loop(0, C)
        def _scale(i):
            for j in range(D // LANES):
                sl = pl.ds(j * LANES, LANES)
                out_v[p, i, sl] = acc_v[p, i, sl] * inv
                acc_v[p, i, sl] = zeros

        out_d[g] = pltpu.async_copy(
            out_v.at[p], out_hbm.at[pl.ds((base + g) * C, C)], osem)

    load_idx(0)
    fire_gathers(0)
    if K > 1:
        load_idx(1)
    for g in range(K):
        if g + 1 < K:
            fire_gathers(g + 1)
        for cp in gat_d[g]:
            cp.wait()
        if g + 2 < K:
            load_idx(g + 2)
        compute_store(g)
    if K >= 2:
        out_d[K - 2].wait()
    out_d[K - 1].wait()


def _body(feat_hbm, idx3_hbm, out_hbm, idx_v, acc_v, out_v, isem, gsem, osem):
    cid = lax.axis_index("c")
    sid = lax.axis_index("s")
    zeros = jnp.zeros((LANES,), jnp.float32)

    @pl.loop(0, C)
    def _zero(i):
        for j in range(D // LANES):
            sl = pl.ds(j * LANES, LANES)
            acc_v[0, i, sl] = zeros
            acc_v[1, i, sl] = zeros

    args = (feat_hbm, idx3_hbm, out_hbm, idx_v, acc_v, out_v, isem, gsem,
            osem)

    @pl.when(cid == 0)
    def _core0():
        _worker(*args, sid * K0, K0)

    @pl.when(cid == 1)
    def _core1():
        _worker(*args, N_SUBCORES * K0 + sid * K1, K1)


@jax.jit
def _gather_mean(features, idx3):
    mesh = plsc.VectorSubcoreMesh(core_axis_name="c", subcore_axis_name="s")
    kfn = pl.kernel(
        _body,
        out_type=jax.ShapeDtypeStruct((K_TOT * C, D), jnp.float32),
        mesh=mesh,
        scratch_types=[
            pltpu.VMEM((2, S, C), jnp.int32),
            pltpu.VMEM((2, C, D), jnp.float32),
            pltpu.VMEM((2, C, D), jnp.float32),
            pltpu.SemaphoreType.DMA,
            pltpu.SemaphoreType.DMA,
            pltpu.SemaphoreType.DMA,
        ],
    )
    return kfn(features, idx3)


def kernel(features, neigh_idx):
    b = neigh_idx.shape[0]
    b_pad = K_TOT * C
    idx = neigh_idx.astype(jnp.int32)
    idx = jnp.pad(idx, ((0, b_pad - b), (0, 0)))
    idx3 = idx.reshape(b_pad // C, C, S).transpose(0, 2, 1)
    out = _gather_mean(features, idx3)
    return out[:b]


# speedup vs baseline: 1.0402x; 1.0402x over previous
import functools

import jax
import jax.numpy as jnp
from jax import lax
from jax.experimental import pallas as pl
from jax.experimental.pallas import tpu as pltpu
from jax.experimental.pallas import tpu_sc as plsc

N_CORES = 2
N_SUBCORES = 16
C = 112
S = 10
D = 128
LANES = 16
K0 = 17
K1 = 11
K_TOT = N_SUBCORES * (K0 + K1)


def _worker(feat_hbm, idx3_hbm, out_hbm, idx_v, acc_v, out_v, isem, gsem,
            osem, base, K):
    zeros = jnp.zeros((LANES,), jnp.float32)
    inv = jnp.full((LANES,), 1.0 / S, jnp.float32)

    idx_d = [None] * K
    gat_d = [None] * K
    out_d = [None] * K

    def load_idx(g):
        idx_d[g] = pltpu.async_copy(idx3_hbm.at[base + g], idx_v.at[g % 2],
                                    isem)

    def fire_gathers(g):
        idx_d[g].wait()
        gat_d[g] = [
            pltpu.async_copy(feat_hbm.at[idx_v.at[g % 2].at[si]],
                             acc_v.at[g % 2], gsem, add=True)
            for si in range(S)
        ]

    def compute_store(g):
        p = g % 2
        if g >= 2:
            out_d[g - 2].wait()

        @pl.loop(0, C)
        def _scale(i):
            for j in range(D // LANES):
                sl = pl.ds(j * LANES, LANES)
                out_v[p, i, sl] = acc_v[p, i, sl] * inv
                acc_v[p, i, sl] = zeros

        out_d[g] = pltpu.async_copy(
            out_v.at[p], out_hbm.at[pl.ds((base + g) * C, C)], osem)

    load_idx(0)
    fire_gathers(0)
    if K > 1:
        load_idx(1)
    for g in range(K):
        if g + 1 < K:
            fire_gathers(g + 1)
        for cp in gat_d[g]:
            cp.wait()
        if g + 2 < K:
            load_idx(g + 2)
        compute_store(g)
    if K >= 2:
        out_d[K - 2].wait()
    out_d[K - 1].wait()


def _body(feat_hbm, idx3_hbm, out_hbm, idx_v, acc_v, out_v, isem, gsem, osem):
    cid = lax.axis_index("c")
    sid = lax.axis_index("s")
    zeros = jnp.zeros((LANES,), jnp.float32)

    @pl.loop(0, C)
    def _zero(i):
        for j in range(D // LANES):
            sl = pl.ds(j * LANES, LANES)
            acc_v[0, i, sl] = zeros
            acc_v[1, i, sl] = zeros

    args = (feat_hbm, idx3_hbm, out_hbm, idx_v, acc_v, out_v, isem, gsem,
            osem)

    @pl.when(cid == 0)
    def _core0():
        _worker(*args, sid * K0, K0)

    @pl.when(cid == 1)
    def _core1():
        _worker(*args, N_SUBCORES * K0 + sid * K1, K1)


@jax.jit
def _gather_mean(features, idx3):
    mesh = plsc.VectorSubcoreMesh(core_axis_name="c", subcore_axis_name="s")
    kfn = pl.kernel(
        _body,
        out_type=jax.ShapeDtypeStruct((K_TOT * C, D), jnp.float32),
        mesh=mesh,
        scratch_types=[
            pltpu.VMEM((2, S, C), jnp.int32),
            pltpu.VMEM((2, C, D), jnp.float32),
            pltpu.VMEM((2, C, D), jnp.float32),
            pltpu.SemaphoreType.DMA,
            pltpu.SemaphoreType.DMA,
            pltpu.SemaphoreType.DMA,
        ],
    )
    return kfn(features, idx3)


def kernel(features, neigh_idx):
    b = neigh_idx.shape[0]
    b_pad = K_TOT * C
    idx = neigh_idx.astype(jnp.int32)
    idx = jnp.pad(idx, ((0, b_pad - b), (0, 0)))
    idx3 = idx.reshape(b_pad // C, C, S).transpose(0, 2, 1)
    out = _gather_mean(features, idx3)
    return out[:b]


# speedup vs baseline: 1.7003x; 1.6346x over previous
import jax
import jax.numpy as jnp
from jax import lax
from jax.experimental import pallas as pl
from jax.experimental.pallas import tpu as pltpu
from jax.experimental.pallas import tpu_sc as plsc

N_CORES = 2
N_SUBCORES = 16
NW = N_CORES * N_SUBCORES
C = 120
S = 10
D = 128
LANES = 16
K = 13
B_MAIN = NW * K * C
CT = 80


def _scale_pass(acc_v, out_v, p, n_rows):
    zeros = jnp.zeros((LANES,), jnp.float32)
    inv = jnp.full((LANES,), 1.0 / S, jnp.float32)

    @pl.loop(0, n_rows)
    def _scale(i):
        for j in range(D // LANES):
            sl = pl.ds(j * LANES, LANES)
            out_v[p, i, sl] = acc_v[p, i, sl] * inv
            acc_v[p, i, sl] = zeros


def _body(feat_hbm, idx3_hbm, idxt_hbm, out_hbm, idx_v, idxt_v, acc_v, out_v,
          isem, gsem, osem):
    cid = lax.axis_index("c")
    sid = lax.axis_index("s")
    wid = sid * N_CORES + cid
    base = wid * K
    zeros = jnp.zeros((LANES,), jnp.float32)

    @pl.loop(0, C)
    def _zero(i):
        for j in range(D // LANES):
            sl = pl.ds(j * LANES, LANES)
            acc_v[0, i, sl] = zeros
            acc_v[1, i, sl] = zeros

    idx_d = [None] * K
    gat_d = [None] * K
    out_d = [None] * K

    def load_idx(g):
        idx_d[g] = pltpu.async_copy(idx3_hbm.at[base + g], idx_v.at[g % 2],
                                    isem)

    def fire_gathers(g):
        idx_d[g].wait()
        gat_d[g] = [
            pltpu.async_copy(feat_hbm.at[idx_v.at[g % 2].at[si]],
                             acc_v.at[g % 2], gsem, add=True)
            for si in range(S)
        ]

    def compute_store(g):
        p = g % 2
        if g >= 2:
            out_d[g - 2].wait()
        _scale_pass(acc_v, out_v, p, C)
        out_d[g] = pltpu.async_copy(
            out_v.at[p], out_hbm.at[pl.ds((base + g) * C, C)], osem)

    load_idx(0)
    fire_gathers(0)
    load_idx(1)
    for g in range(K):
        if g + 1 < K:
            fire_gathers(g + 1)
        for cp in gat_d[g]:
            cp.wait()
        if g + 2 < K:
            load_idx(g + 2)
        compute_store(g)

    @pl.when(wid == NW - 1)
    def _tail():
        pltpu.sync_copy(idxt_hbm, idxt_v)
        tg = [
            pltpu.async_copy(feat_hbm.at[idxt_v.at[si]],
                             acc_v.at[0].at[pl.ds(0, CT)], gsem, add=True)
            for si in range(S)
        ]
        for cp in tg:
            cp.wait()
        _scale_pass(acc_v, out_v, 0, CT)
        pltpu.sync_copy(out_v.at[0].at[pl.ds(0, CT)],
                        out_hbm.at[pl.ds(B_MAIN, CT)])

    out_d[K - 2].wait()
    out_d[K - 1].wait()


@jax.jit
def _gather_mean(features, idx3, idxt):
    mesh = plsc.VectorSubcoreMesh(core_axis_name="c", subcore_axis_name="s")
    kfn = pl.kernel(
        _body,
        out_type=jax.ShapeDtypeStruct((B_MAIN + CT, D), jnp.float32),
        mesh=mesh,
        scratch_types=[
            pltpu.VMEM((2, S, C), jnp.int32),
            pltpu.VMEM((S, CT), jnp.int32),
            pltpu.VMEM((2, C, D), jnp.float32),
            pltpu.VMEM((2, C, D), jnp.float32),
            pltpu.SemaphoreType.DMA,
            pltpu.SemaphoreType.DMA,
            pltpu.SemaphoreType.DMA,
        ],
    )
    return kfn(features, idx3, idxt)


def kernel(features, neigh_idx):
    b = neigh_idx.shape[0]
    assert b == B_MAIN + CT
    idx = neigh_idx.astype(jnp.int32)
    idx3 = idx[:B_MAIN].reshape(B_MAIN // C, C, S).transpose(0, 2, 1)
    idxt = idx[B_MAIN:].T
    return _gather_mean(features, idx3, idxt)


# speedup vs baseline: 1.7201x; 1.0116x over previous
import jax
import jax.numpy as jnp
from jax import lax
from jax.experimental import pallas as pl
from jax.experimental.pallas import tpu as pltpu
from jax.experimental.pallas import tpu_sc as plsc

N_CORES = 2
N_SUBCORES = 16
NW = N_CORES * N_SUBCORES
C = 120
S = 10
D = 128
LANES = 16
K = 13
B_MAIN = NW * K * C
TW = 10
CM = 8


def _scale_pass(acc_v, out_v, p, n_rows):
    zeros = jnp.zeros((LANES,), jnp.float32)
    inv = jnp.full((LANES,), 1.0 / S, jnp.float32)

    @pl.loop(0, n_rows)
    def _scale(i):
        for j in range(D // LANES):
            sl = pl.ds(j * LANES, LANES)
            out_v[p, i, sl] = acc_v[p, i, sl] * inv
            acc_v[p, i, sl] = zeros


def _body(feat_hbm, idx3_hbm, idxt_hbm, out_hbm, idx_v, idxt_v, acc_v, out_v,
          acc_t, out_t, isem, gsem, osem, tsem):
    cid = lax.axis_index("c")
    sid = lax.axis_index("s")
    wid = sid * N_CORES + cid
    base = wid * K
    is_tail = wid >= NW - TW
    t = jnp.maximum(wid - (NW - TW), 0)
    zeros = jnp.zeros((LANES,), jnp.float32)

    @pl.loop(0, C)
    def _zero(i):
        for j in range(D // LANES):
            sl = pl.ds(j * LANES, LANES)
            acc_v[0, i, sl] = zeros
            acc_v[1, i, sl] = zeros

    @pl.loop(0, CM)
    def _zero_t(i):
        for j in range(D // LANES):
            acc_t[0, i, pl.ds(j * LANES, LANES)] = zeros

    idx_d = [None] * K
    gat_d = [None] * K
    out_d = [None] * K

    def load_idx(g):
        idx_d[g] = pltpu.async_copy(idx3_hbm.at[base + g], idx_v.at[g % 2],
                                    isem)

    def fire_gathers(g):
        idx_d[g].wait()
        gat_d[g] = [
            pltpu.async_copy(feat_hbm.at[idx_v.at[g % 2].at[si]],
                             acc_v.at[g % 2], gsem, add=True)
            for si in range(S)
        ]

    def compute_store(g):
        p = g % 2
        if g >= 2:
            out_d[g - 2].wait()
        _scale_pass(acc_v, out_v, p, C)
        out_d[g] = pltpu.async_copy(
            out_v.at[p], out_hbm.at[pl.ds((base + g) * C, C)], osem)

    load_idx(0)
    fire_gathers(0)
    load_idx(1)
    for g in range(K):
        if g + 1 < K:
            fire_gathers(g + 1)
        for cp in gat_d[g]:
            cp.wait()
        if g + 2 < K:
            load_idx(g + 2)
        compute_store(g)
        if g == 0:
            @pl.when(is_tail)
            def _tail():
                pltpu.sync_copy(idxt_hbm.at[t], idxt_v)
                tg = [
                    pltpu.async_copy(feat_hbm.at[idxt_v.at[si]],
                                     acc_t.at[0], tsem, add=True)
                    for si in range(S)
                ]
                for cp in tg:
                    cp.wait()
                _scale_pass(acc_t, out_t, 0, CM)
                pltpu.sync_copy(out_t.at[0],
                                out_hbm.at[pl.ds(B_MAIN + t * CM, CM)])

    out_d[K - 2].wait()
    out_d[K - 1].wait()


@jax.jit
def _gather_mean(features, idx3, idxt):
    mesh = plsc.VectorSubcoreMesh(core_axis_name="c", subcore_axis_name="s")
    kfn = pl.kernel(
        _body,
        out_type=jax.ShapeDtypeStruct((B_MAIN + TW * CM, D), jnp.float32),
        mesh=mesh,
        scratch_types=[
            pltpu.VMEM((2, S, C), jnp.int32),
            pltpu.VMEM((S, CM), jnp.int32),
            pltpu.VMEM((2, C, D), jnp.float32),
            pltpu.VMEM((2, C, D), jnp.float32),
            pltpu.VMEM((1, CM, D), jnp.float32),
            pltpu.VMEM((1, CM, D), jnp.float32),
            pltpu.SemaphoreType.DMA,
            pltpu.SemaphoreType.DMA,
            pltpu.SemaphoreType.DMA,
            pltpu.SemaphoreType.DMA,
        ],
    )
    return kfn(features, idx3, idxt)


def kernel(features, neigh_idx):
    b = neigh_idx.shape[0]
    assert b == B_MAIN + TW * CM
    idx = neigh_idx.astype(jnp.int32)
    idx3 = idx[:B_MAIN].reshape(B_MAIN // C, C, S).transpose(0, 2, 1)
    idxt3 = idx[B_MAIN:].reshape(TW, CM, S).transpose(0, 2, 1)
    return _gather_mean(features, idx3, idxt3)
